# baseline (speedup 1.0000x reference)
import functools

import jax
import jax.numpy as jnp
from jax import lax
from jax.experimental import pallas as pl
from jax.experimental.pallas import tpu as pltpu
from jax.experimental.pallas import tpu_sc as plsc

VOCAB = 1000000
EMB = 64
B = 16384
L = 50

NC = 2
NS = 16
NW = NC * NS

VC = 256
N_FULL = VOCAB // VC
TAIL0 = N_FULL * VC
TAIL_W = N_FULL % NW


def _tr_body(tabT_hbm, tail2_hbm, t2_hbm, in_v, out_v, isem, osem):
    wid = lax.axis_index("s") * NC + lax.axis_index("c")
    n_my = (N_FULL - 1 - wid) // NW + 1

    iota = lax.iota(jnp.int32, 16)
    e_vecs = [iota + 16 * k for k in range(4)]

    def swizzle_rows(n_rows):
        def row_body(s, carry2):
            u = jnp.full((16,), s, jnp.int32)
            for k in range(4):
                g = plsc.load_gather(in_v, [e_vecs[k], u])
                out_v[s, pl.ds(16 * k, 16)] = g
                out_v[s, pl.ds(64 + 16 * k, 16)] = g
            return carry2

        lax.fori_loop(0, n_rows, row_body, 0)

    def chunk_body(i, carry):
        c = wid + i * NW
        v0 = pl.multiple_of(c * VC, 128)
        pltpu.async_copy(tabT_hbm.at[:, pl.ds(v0, VC)], in_v, isem).wait()
        swizzle_rows(VC)
        pltpu.async_copy(out_v, t2_hbm.at[pl.ds(v0, VC)], osem).wait()
        return carry

    lax.fori_loop(0, n_my, chunk_body, 0)

    @pl.when(wid == TAIL_W)
    def _():
        tail = VOCAB - TAIL0
        pltpu.async_copy(tail2_hbm, out_v.at[pl.ds(0, tail)], isem).wait()
        pltpu.async_copy(out_v.at[pl.ds(0, tail)],
                         t2_hbm.at[pl.ds(TAIL0, tail)], osem).wait()


_relayout = functools.partial(
    pl.kernel,
    mesh=plsc.VectorSubcoreMesh(core_axis_name="c", subcore_axis_name="s"),
    out_type=jax.ShapeDtypeStruct((VOCAB, 128), jnp.float32),
    scratch_types=[
        pltpu.VMEM((EMB, VC), jnp.float32),
        pltpu.VMEM((VC, 128), jnp.float32),
        pltpu.SemaphoreType.DMA,
        pltpu.SemaphoreType.DMA,
    ],
    compiler_params=pltpu.CompilerParams(needs_layout_passes=False),
)(_tr_body)

BAGS_PER_W = B // NW
CHUNK = 8
N_CHUNKS = BAGS_PER_W // CHUNK
IDX_PER_CHUNK = CHUNK * L
VPR = EMB // 16


def _emb_body(idx_hbm, t2_hbm, out_hbm, idx_v, rows0_v, rows1_v, acc_v,
              sem0, sem1):
    wid = lax.axis_index("s") * NC + lax.axis_index("c")
    bag0 = wid * BAGS_PER_W
    IDX_PER_W = BAGS_PER_W * L

    pltpu.sync_copy(idx_hbm.at[pl.ds(wid * IDX_PER_W, IDX_PER_W)], idx_v)

    rows = (rows0_v, rows1_v)
    sems = (sem0, sem1)

    def idx_at(c):
        return idx_v.at[pl.ds(c * IDX_PER_CHUNK, IDX_PER_CHUNK)]

    def start(c, b):
        return pltpu.async_copy(t2_hbm.at[idx_at(c)], rows[b], sems[b])

    def reduce_chunk(c, b):
        rows_v = rows[b]
        base_bag = bag0 + c * CHUNK

        def bag_body(j, carry2):
            base = j * L
            accs = tuple(rows_v[base, pl.ds(16 * k, 16)]
                         for k in range(VPR))
            for i in range(1, L):
                accs = tuple(a + rows_v[base + i, pl.ds(16 * k, 16)]
                             for k, a in enumerate(accs))
            for k in range(VPR):
                acc_v[j, pl.ds(16 * k, 16)] = accs[k] * (1.0 / L)
            return carry2

        lax.fori_loop(0, CHUNK, bag_body, 0)
        pltpu.sync_copy(acc_v, out_hbm.at[pl.ds(base_bag, CHUNK)])

    start(0, 0)

    def pair_body(p, carry):
        c0 = 2 * p
        start(c0 + 1, 1)
        pltpu.make_async_copy(t2_hbm.at[idx_at(0)], rows[0], sems[0]).wait()
        reduce_chunk(c0, 0)

        @pl.when(c0 + 2 < N_CHUNKS)
        def _():
            start(c0 + 2, 0)

        pltpu.make_async_copy(t2_hbm.at[idx_at(0)], rows[1], sems[1]).wait()
        reduce_chunk(c0 + 1, 1)
        return carry

    lax.fori_loop(0, N_CHUNKS // 2, pair_body, 0)


_embed_bag = functools.partial(
    pl.kernel,
    mesh=plsc.VectorSubcoreMesh(core_axis_name="c", subcore_axis_name="s"),
    out_type=jax.ShapeDtypeStruct((B, EMB), jnp.float32),
    scratch_types=[
        pltpu.VMEM((BAGS_PER_W * L,), jnp.int32),
        pltpu.VMEM((IDX_PER_CHUNK, 128), jnp.float32),
        pltpu.VMEM((IDX_PER_CHUNK, 128), jnp.float32),
        pltpu.VMEM((CHUNK, EMB), jnp.float32),
        pltpu.SemaphoreType.DMA,
        pltpu.SemaphoreType.DMA,
    ],
)(_emb_body)


def _mlp_body(e_ref, w1_ref, b1_ref, w2_ref, b2_ref, o_ref):
    h = jnp.dot(e_ref[...], w1_ref[...],
                preferred_element_type=jnp.float32) + b1_ref[...]
    h = jnp.maximum(h, 0.0)
    z = jnp.sum(h * w2_ref[...], axis=1, keepdims=True) + b2_ref[...]
    o_ref[...] = 1.0 / (1.0 + jnp.exp(-z))


_N_BLOCKS = 8
_BLK = B // _N_BLOCKS

_mlp = pl.pallas_call(
    _mlp_body,
    grid=(_N_BLOCKS,),
    in_specs=[
        pl.BlockSpec((_BLK, EMB), lambda i: (i, 0)),
        pl.BlockSpec((EMB, 128), lambda i: (0, 0)),
        pl.BlockSpec((1, 128), lambda i: (0, 0)),
        pl.BlockSpec((1, 128), lambda i: (0, 0)),
        pl.BlockSpec((1, 1), lambda i: (0, 0)),
    ],
    out_specs=pl.BlockSpec((_BLK, 1), lambda i: (i, 0)),
    out_shape=jax.ShapeDtypeStruct((B, 1), jnp.float32),
)


def kernel(x, emb_table, W1, b1, W2, b2):
    tail2 = jnp.tile(emb_table[TAIL0:], (1, 2))
    t2 = _relayout(emb_table.T, tail2)
    embeds = _embed_bag(x.reshape(-1), t2)
    return _mlp(embeds, W1, b1.reshape(1, 128), W2.reshape(1, 128),
                b2.reshape(1, 1))


# speedup vs baseline: 1.2383x; 1.2383x over previous
import functools

import jax
import jax.numpy as jnp
from jax import lax
from jax.experimental import pallas as pl
from jax.experimental.pallas import tpu as pltpu
from jax.experimental.pallas import tpu_sc as plsc

VOCAB = 1000000
EMB = 64
B = 16384
L = 50

NC = 2
NS = 16
NW = NC * NS

VC = 256
N_FULL = VOCAB // VC
TAIL0 = N_FULL * VC
TAIL_W = N_FULL % NW


UNROLL = 8


def _tr_body(tabT_hbm, tail2_hbm, t2_hbm, in0_v, in1_v, out0_v, out1_v,
             isem0, isem1, osem0, osem1):
    wid = lax.axis_index("s") * NC + lax.axis_index("c")
    n_my = (N_FULL - 1 - wid) // NW + 1

    iota = lax.iota(jnp.int32, 16)
    e_vecs = [iota + 16 * k for k in range(4)]
    ins = (in0_v, in1_v)
    outs = (out0_v, out1_v)
    isems = (isem0, isem1)
    osems = (osem0, osem1)

    def v0_of(c):
        return pl.multiple_of(c * VC, 128)

    def start_in(c, b):
        pltpu.async_copy(tabT_hbm.at[:, pl.ds(v0_of(c), VC)], ins[b],
                         isems[b])

    def wait_in(b):
        pltpu.make_async_copy(tabT_hbm.at[:, pl.ds(0, VC)], ins[b],
                              isems[b]).wait()

    def start_out(c, b):
        pltpu.async_copy(outs[b], t2_hbm.at[pl.ds(v0_of(c), VC)], osems[b])

    def wait_out(b):
        pltpu.make_async_copy(outs[b], t2_hbm.at[pl.ds(0, VC)],
                              osems[b]).wait()

    def swizzle(b):
        in_v, out_v = ins[b], outs[b]

        def row_body(s8, carry2):
            s0 = s8 * UNROLL
            for d in range(UNROLL):
                u = jnp.full((16,), s0 + d, jnp.int32)
                for k in range(4):
                    out_v[s0 + d, pl.ds(16 * k, 16)] = (
                        plsc.load_gather(in_v, [e_vecs[k], u]))
            return carry2

        lax.fori_loop(0, VC // UNROLL, row_body, 0)

    start_in(wid, 0)

    def pair_body(p, carry):
        c0 = wid + 2 * p * NW
        c1 = c0 + NW

        @pl.when(c1 < N_FULL)
        def _():
            start_in(c1, 1)

        wait_in(0)

        @pl.when(p > 0)
        def _():
            wait_out(0)

        swizzle(0)
        start_out(c0, 0)

        @pl.when(c1 < N_FULL)
        def _():
            @pl.when(c1 + NW < N_FULL)
            def _():
                start_in(c1 + NW, 0)

            wait_in(1)

            @pl.when(p > 0)
            def _():
                wait_out(1)

            swizzle(1)
            start_out(c1, 1)

        return carry

    n_pairs = (n_my + 1) // 2
    lax.fori_loop(0, n_pairs, pair_body, 0)
    wait_out(0)

    @pl.when(n_my >= 2)
    def _():
        wait_out(1)

    @pl.when(wid == TAIL_W)
    def _():
        tail = VOCAB - TAIL0
        pltpu.async_copy(tail2_hbm, outs[0].at[pl.ds(0, tail)],
                         isems[0]).wait()
        pltpu.async_copy(outs[0].at[pl.ds(0, tail)],
                         t2_hbm.at[pl.ds(TAIL0, tail)], osems[0]).wait()


_relayout = functools.partial(
    pl.kernel,
    mesh=plsc.VectorSubcoreMesh(core_axis_name="c", subcore_axis_name="s"),
    out_type=jax.ShapeDtypeStruct((VOCAB, 128), jnp.float32),
    scratch_types=[
        pltpu.VMEM((EMB, VC), jnp.float32),
        pltpu.VMEM((EMB, VC), jnp.float32),
        pltpu.VMEM((VC, 128), jnp.float32),
        pltpu.VMEM((VC, 128), jnp.float32),
        pltpu.SemaphoreType.DMA,
        pltpu.SemaphoreType.DMA,
        pltpu.SemaphoreType.DMA,
        pltpu.SemaphoreType.DMA,
    ],
    compiler_params=pltpu.CompilerParams(needs_layout_passes=False),
)(_tr_body)

BAGS_PER_W = B // NW
CHUNK = 8
N_CHUNKS = BAGS_PER_W // CHUNK
IDX_PER_CHUNK = CHUNK * L
VPR = EMB // 16


def _emb_body(idx_hbm, t2_hbm, out_hbm, idx_v, rows0_v, rows1_v, acc_v,
              sem0, sem1):
    wid = lax.axis_index("s") * NC + lax.axis_index("c")
    bag0 = wid * BAGS_PER_W
    IDX_PER_W = BAGS_PER_W * L

    pltpu.sync_copy(idx_hbm.at[pl.ds(wid * IDX_PER_W, IDX_PER_W)], idx_v)

    rows = (rows0_v, rows1_v)
    sems = (sem0, sem1)

    def idx_at(c):
        return idx_v.at[pl.ds(c * IDX_PER_CHUNK, IDX_PER_CHUNK)]

    def start(c, b):
        return pltpu.async_copy(t2_hbm.at[idx_at(c)], rows[b], sems[b])

    def reduce_chunk(c, b):
        rows_v = rows[b]
        base_bag = bag0 + c * CHUNK

        def bag_body(j, carry2):
            base = j * L
            accs = tuple(rows_v[base, pl.ds(16 * k, 16)]
                         for k in range(VPR))
            for i in range(1, L):
                accs = tuple(a + rows_v[base + i, pl.ds(16 * k, 16)]
                             for k, a in enumerate(accs))
            for k in range(VPR):
                acc_v[j, pl.ds(16 * k, 16)] = accs[k] * (1.0 / L)
            return carry2

        lax.fori_loop(0, CHUNK, bag_body, 0)
        pltpu.sync_copy(acc_v, out_hbm.at[pl.ds(base_bag, CHUNK)])

    start(0, 0)

    def pair_body(p, carry):
        c0 = 2 * p
        start(c0 + 1, 1)
        pltpu.make_async_copy(t2_hbm.at[idx_at(0)], rows[0], sems[0]).wait()
        reduce_chunk(c0, 0)

        @pl.when(c0 + 2 < N_CHUNKS)
        def _():
            start(c0 + 2, 0)

        pltpu.make_async_copy(t2_hbm.at[idx_at(0)], rows[1], sems[1]).wait()
        reduce_chunk(c0 + 1, 1)
        return carry

    lax.fori_loop(0, N_CHUNKS // 2, pair_body, 0)


_embed_bag = functools.partial(
    pl.kernel,
    mesh=plsc.VectorSubcoreMesh(core_axis_name="c", subcore_axis_name="s"),
    out_type=jax.ShapeDtypeStruct((B, EMB), jnp.float32),
    scratch_types=[
        pltpu.VMEM((BAGS_PER_W * L,), jnp.int32),
        pltpu.VMEM((IDX_PER_CHUNK, 128), jnp.float32),
        pltpu.VMEM((IDX_PER_CHUNK, 128), jnp.float32),
        pltpu.VMEM((CHUNK, EMB), jnp.float32),
        pltpu.SemaphoreType.DMA,
        pltpu.SemaphoreType.DMA,
    ],
)(_emb_body)


def _mlp_body(e_ref, w1_ref, b1_ref, w2_ref, b2_ref, o_ref):
    h = jnp.dot(e_ref[...], w1_ref[...],
                preferred_element_type=jnp.float32) + b1_ref[...]
    h = jnp.maximum(h, 0.0)
    z = jnp.sum(h * w2_ref[...], axis=1, keepdims=True) + b2_ref[...]
    o_ref[...] = 1.0 / (1.0 + jnp.exp(-z))


_N_BLOCKS = 8
_BLK = B // _N_BLOCKS

_mlp = pl.pallas_call(
    _mlp_body,
    grid=(_N_BLOCKS,),
    in_specs=[
        pl.BlockSpec((_BLK, EMB), lambda i: (i, 0)),
        pl.BlockSpec((EMB, 128), lambda i: (0, 0)),
        pl.BlockSpec((1, 128), lambda i: (0, 0)),
        pl.BlockSpec((1, 128), lambda i: (0, 0)),
        pl.BlockSpec((1, 1), lambda i: (0, 0)),
    ],
    out_specs=pl.BlockSpec((_BLK, 1), lambda i: (i, 0)),
    out_shape=jax.ShapeDtypeStruct((B, 1), jnp.float32),
)


def kernel(x, emb_table, W1, b1, W2, b2):
    tail2 = jnp.tile(emb_table[TAIL0:], (1, 2))
    t2 = _relayout(emb_table.T, tail2)
    embeds = _embed_bag(x.reshape(-1), t2)
    return _mlp(embeds, W1, b1.reshape(1, 128), W2.reshape(1, 128),
                b2.reshape(1, 1))


# speedup vs baseline: 2.9192x; 2.3575x over previous
import functools

import jax
import jax.numpy as jnp
from jax import lax
from jax.experimental import pallas as pl
from jax.experimental.pallas import tpu as pltpu
from jax.experimental.pallas import tpu_sc as plsc

EMB = 64
B = 16384
L = 50

NC = 2
NS = 16
NW = NC * NS
BAGS_PER_W = B // NW
CHUNK = 16
N_CHUNKS = BAGS_PER_W // CHUNK
IDX_PER_CHUNK = CHUNK * L
VPR = EMB // 16


def _emb_body(x_hbm, tab_hbm, out_hbm, idx_v, rows0_v, rows1_v, acc_v,
              sem0, sem1):
    wid = lax.axis_index("s") * NC + lax.axis_index("c")
    bag0 = wid * BAGS_PER_W
    chunk0 = wid * N_CHUNKS

    pltpu.sync_copy(x_hbm.at[pl.ds(chunk0, N_CHUNKS)], idx_v)

    rows = (rows0_v, rows1_v)
    sems = (sem0, sem1)

    def start(c, b):
        return pltpu.async_copy(tab_hbm.at[idx_v.at[c]], rows[b], sems[b])

    def reduce_chunk(c, b):
        rows_v = rows[b]
        base_bag = bag0 + c * CHUNK

        def bag_body(j, carry2):
            base = j * L
            accs = tuple(rows_v[base, pl.ds(16 * k, 16)]
                         for k in range(VPR))
            for i in range(1, L):
                accs = tuple(a + rows_v[base + i, pl.ds(16 * k, 16)]
                             for k, a in enumerate(accs))
            for k in range(VPR):
                acc_v[j, pl.ds(16 * k, 16)] = accs[k] * (1.0 / L)
            return carry2

        lax.fori_loop(0, CHUNK, bag_body, 0)
        pltpu.sync_copy(acc_v, out_hbm.at[pl.ds(base_bag, CHUNK)])

    start(0, 0)

    def pair_body(p, carry):
        c0 = 2 * p
        start(c0 + 1, 1)
        pltpu.make_async_copy(tab_hbm.at[idx_v.at[c0]], rows[0],
                              sems[0]).wait()
        reduce_chunk(c0, 0)

        @pl.when(c0 + 2 < N_CHUNKS)
        def _():
            start(c0 + 2, 0)

        pltpu.make_async_copy(tab_hbm.at[idx_v.at[c0 + 1]], rows[1],
                              sems[1]).wait()
        reduce_chunk(c0 + 1, 1)
        return carry

    lax.fori_loop(0, N_CHUNKS // 2, pair_body, 0)


_embed_bag = functools.partial(
    pl.kernel,
    mesh=plsc.VectorSubcoreMesh(core_axis_name="c", subcore_axis_name="s"),
    out_type=jax.ShapeDtypeStruct((B, EMB), jnp.float32),
    scratch_types=[
        pltpu.VMEM((N_CHUNKS, IDX_PER_CHUNK), jnp.int32),
        pltpu.VMEM((IDX_PER_CHUNK, EMB), jnp.float32),
        pltpu.VMEM((IDX_PER_CHUNK, EMB), jnp.float32),
        pltpu.VMEM((CHUNK, EMB), jnp.float32),
        pltpu.SemaphoreType.DMA,
        pltpu.SemaphoreType.DMA,
    ],
    compiler_params=pltpu.CompilerParams(use_tc_tiling_on_sc=False),
)(_emb_body)


def _mlp_body(e_ref, w1_ref, b1_ref, w2_ref, b2_ref, o_ref):
    h = jnp.dot(e_ref[...], w1_ref[...],
                preferred_element_type=jnp.float32) + b1_ref[...]
    h = jnp.maximum(h, 0.0)
    z = jnp.sum(h * w2_ref[...], axis=1, keepdims=True) + b2_ref[...]
    o_ref[...] = 1.0 / (1.0 + jnp.exp(-z))


_N_BLOCKS = 8
_BLK = B // _N_BLOCKS

_mlp = pl.pallas_call(
    _mlp_body,
    grid=(_N_BLOCKS,),
    in_specs=[
        pl.BlockSpec((_BLK, EMB), lambda i: (i, 0)),
        pl.BlockSpec((EMB, 128), lambda i: (0, 0)),
        pl.BlockSpec((1, 128), lambda i: (0, 0)),
        pl.BlockSpec((1, 128), lambda i: (0, 0)),
        pl.BlockSpec((1, 1), lambda i: (0, 0)),
    ],
    out_specs=pl.BlockSpec((_BLK, 1), lambda i: (i, 0)),
    out_shape=jax.ShapeDtypeStruct((B, 1), jnp.float32),
)


def kernel(x, emb_table, W1, b1, W2, b2):
    embeds = _embed_bag(x.reshape(B // CHUNK, IDX_PER_CHUNK), emb_table)
    return _mlp(embeds, W1, b1.reshape(1, 128), W2.reshape(1, 128),
                b2.reshape(1, 1))
